# baseline (speedup 1.0000x reference)
import jax
import jax.numpy as jnp
from jax import lax
from jax.experimental import pallas as pl
from jax.experimental.pallas import tpu as pltpu
from jax.experimental.pallas import tpu_sc as plsc

B, N, D = 4, 8192, 2048
H = N // 2
NC, NS = 2, 16
NW = NC * NS
WPB = NW // B
ROWS_PER_W = H // WPB
R = 8


def _body(in_hbm, out_hbm, buf0, buf1, r0s, r1s, w0s, w1s):
    wid = lax.axis_index("s") * NC + lax.axis_index("c")
    b = wid // WPB
    q = wid % WPB
    nsteps = ROWS_PER_W // R
    nhalf = nsteps // 2

    def rd(it, buf, sem):
        r0 = q * ROWS_PER_W + it * R
        pltpu.async_copy(in_hbm.at[b, pl.ds(r0, R), :], buf.at[:, 0, :], sem)
        pltpu.async_copy(in_hbm.at[b, pl.ds(H + r0, R), :], buf.at[:, 1, :],
                         sem)

    def wr(it, buf, sem):
        r0 = q * ROWS_PER_W + it * R
        pltpu.async_copy(buf, out_hbm.at[b, pl.ds(r0, R), :, :], sem)

    def wait_rd(buf, sem):
        pltpu.make_async_copy(in_hbm.at[b, pl.ds(0, R), :], buf.at[:, 0, :],
                              sem).wait()
        pltpu.make_async_copy(in_hbm.at[b, pl.ds(0, R), :], buf.at[:, 1, :],
                              sem).wait()

    def wait_wr(buf, sem):
        pltpu.make_async_copy(buf, out_hbm.at[b, pl.ds(0, R), :, :],
                              sem).wait()

    rd(0, buf0, r0s)

    def step(i2, carry):
        it0 = 2 * i2
        wait_rd(buf0, r0s)
        pl.when(i2 > 0)(lambda: wait_wr(buf1, w1s))
        rd(it0 + 1, buf1, r1s)
        wr(it0, buf0, w0s)
        wait_rd(buf1, r1s)

        def refill_buf0():
            wait_wr(buf0, w0s)
            rd(it0 + 2, buf0, r0s)

        pl.when(i2 + 1 < nhalf)(refill_buf0)
        wr(it0 + 1, buf1, w1s)
        return carry

    lax.fori_loop(0, nhalf, step, 0)
    wait_wr(buf0, w0s)
    wait_wr(buf1, w1s)


@jax.jit
def kernel(inputs):
    mesh = plsc.VectorSubcoreMesh(
        core_axis_name="c", subcore_axis_name="s", num_cores=NC,
        num_subcores=NS)
    out4 = pl.kernel(
        _body,
        out_type=jax.ShapeDtypeStruct((B, H, 2, D), jnp.float32),
        mesh=mesh,
        scratch_types=[
            pltpu.VMEM((R, 2, D), jnp.float32),
            pltpu.VMEM((R, 2, D), jnp.float32),
            pltpu.SemaphoreType.DMA,
            pltpu.SemaphoreType.DMA,
            pltpu.SemaphoreType.DMA,
            pltpu.SemaphoreType.DMA,
        ],
    )(inputs)
    return out4.reshape(B, N, D)


# speedup vs baseline: 1.0124x; 1.0124x over previous
import jax
import jax.numpy as jnp
from jax import lax
from jax.experimental import pallas as pl
from jax.experimental.pallas import tpu as pltpu
from jax.experimental.pallas import tpu_sc as plsc

B, N, D = 4, 8192, 2048
H = N // 2
NC, NS = 2, 16
NW = NC * NS
WPB = NW // B
ROWS_PER_W = H // WPB
R = 4
NBUF = 4


def _body(in_hbm, out_hbm, bufs, rsems, wsems):
    wid = lax.axis_index("s") * NC + lax.axis_index("c")
    b = wid // WPB
    q = wid % WPB
    nsteps = ROWS_PER_W // R
    ngroups = nsteps // NBUF

    def rd(it, p):
        r0 = q * ROWS_PER_W + it * R
        pltpu.async_copy(in_hbm.at[b, pl.ds(r0, R), :], bufs.at[p, :, 0, :],
                         rsems.at[p])
        pltpu.async_copy(in_hbm.at[b, pl.ds(H + r0, R), :],
                         bufs.at[p, :, 1, :], rsems.at[p])

    def wr(it, p):
        r0 = q * ROWS_PER_W + it * R
        pltpu.async_copy(bufs.at[p], out_hbm.at[b, pl.ds(r0, R), :, :],
                         wsems.at[p])

    def wait_rd(p):
        pltpu.make_async_copy(in_hbm.at[b, pl.ds(0, R), :],
                              bufs.at[p, :, 0, :], rsems.at[p]).wait()
        pltpu.make_async_copy(in_hbm.at[b, pl.ds(0, R), :],
                              bufs.at[p, :, 1, :], rsems.at[p]).wait()

    def wait_wr(p):
        pltpu.make_async_copy(bufs.at[p], out_hbm.at[b, pl.ds(0, R), :, :],
                              wsems.at[p]).wait()

    for p in range(NBUF):
        rd(p, p)

    def step(g, carry):
        it0 = g * NBUF
        for p in range(NBUF):
            wait_rd(p)
            wr(it0 + p, p)
        def refill():
            for p in range(NBUF):
                wait_wr(p)
                rd(it0 + NBUF + p, p)
        pl.when(g + 1 < ngroups)(refill)
        return carry

    lax.fori_loop(0, ngroups, step, 0)
    for p in range(NBUF):
        wait_wr(p)


@jax.jit
def kernel(inputs):
    mesh = plsc.VectorSubcoreMesh(
        core_axis_name="c", subcore_axis_name="s", num_cores=NC,
        num_subcores=NS)
    out4 = pl.kernel(
        _body,
        out_type=jax.ShapeDtypeStruct((B, H, 2, D), jnp.float32),
        mesh=mesh,
        scratch_types=[
            pltpu.VMEM((NBUF, R, 2, D), jnp.float32),
            pltpu.SemaphoreType.DMA((NBUF,)),
            pltpu.SemaphoreType.DMA((NBUF,)),
        ],
    )(inputs)
    return out4.reshape(B, N, D)


# speedup vs baseline: 2.5161x; 2.4854x over previous
import jax
import jax.numpy as jnp
from jax import lax
from jax.experimental import pallas as pl
from jax.experimental.pallas import tpu as pltpu
from jax.experimental.pallas import tpu_sc as plsc

B, N, D = 4, 8192, 2048
H = N // 2
NC, NS = 2, 16
NW = NC * NS
TOT = B * N
RW = TOT // NW
CH = 16


def _body(in_hbm, out_hbm, buf0, buf1, idx0, idx1, r0s, r1s, w0s, w1s):
    wid = lax.axis_index("s") * NC + lax.axis_index("c")
    o_base = wid * RW
    b = o_base // N
    i0_base = (o_base % N) // 2
    k = lax.iota(jnp.int32, 16)
    pattern = (k >> 1) + (k & 1) * H + b * N
    nsteps = RW // CH
    nhalf = nsteps // 2

    def rd(it, buf, idx, sem):
        idx[...] = pattern + (i0_base + it * (CH // 2))
        pltpu.async_copy(in_hbm.at[idx], buf, sem)

    def wr(it, buf, sem):
        pltpu.async_copy(buf, out_hbm.at[pl.ds(o_base + it * CH, CH), :],
                         sem)

    def wait_rd(buf, idx, sem):
        pltpu.make_async_copy(in_hbm.at[idx], buf, sem).wait()

    def wait_wr(buf, sem):
        pltpu.make_async_copy(buf, out_hbm.at[pl.ds(o_base, CH), :],
                              sem).wait()

    rd(0, buf0, idx0, r0s)

    def step(i2, carry):
        it0 = 2 * i2
        wait_rd(buf0, idx0, r0s)
        pl.when(i2 > 0)(lambda: wait_wr(buf1, w1s))
        rd(it0 + 1, buf1, idx1, r1s)
        wr(it0, buf0, w0s)
        wait_rd(buf1, idx1, r1s)

        def refill_buf0():
            wait_wr(buf0, w0s)
            rd(it0 + 2, buf0, idx0, r0s)

        pl.when(i2 + 1 < nhalf)(refill_buf0)
        wr(it0 + 1, buf1, w1s)
        return carry

    lax.fori_loop(0, nhalf, step, 0)
    wait_wr(buf0, w0s)
    wait_wr(buf1, w1s)


@jax.jit
def kernel(inputs):
    mesh = plsc.VectorSubcoreMesh(
        core_axis_name="c", subcore_axis_name="s", num_cores=NC,
        num_subcores=NS)
    out = pl.kernel(
        _body,
        out_type=jax.ShapeDtypeStruct((TOT, D), jnp.float32),
        mesh=mesh,
        scratch_types=[
            pltpu.VMEM((CH, D), jnp.float32),
            pltpu.VMEM((CH, D), jnp.float32),
            pltpu.VMEM((16,), jnp.int32),
            pltpu.VMEM((16,), jnp.int32),
            pltpu.SemaphoreType.DMA,
            pltpu.SemaphoreType.DMA,
            pltpu.SemaphoreType.DMA,
            pltpu.SemaphoreType.DMA,
        ],
    )(inputs.reshape(TOT, D))
    return out.reshape(B, N, D)
